# baseline (speedup 1.0000x reference)
import functools

import jax
import jax.numpy as jnp
from jax import lax
from jax.experimental import pallas as pl
from jax.experimental.pallas import tpu as pltpu
from jax.experimental.pallas import tpu_sc as plsc

_LANES = 16


def _make_kernel(B, T, V, D, NC, NS, Cq, T_sc):
    NW = NC * NS
    TB = T_sc // NW
    n_chunks = TB // Cq
    mesh = plsc.VectorSubcoreMesh(core_axis_name="c", subcore_axis_name="s")

    @functools.partial(
        pl.kernel,
        mesh=mesh,
        out_type=jax.ShapeDtypeStruct((B, T, D), jnp.float32),
        scratch_types=[
            pltpu.VMEM((B, TB), jnp.int32),
            pltpu.VMEM((2, B, Cq, D), jnp.float32),
            pltpu.VMEM((2, Cq, D), jnp.float32),
            pltpu.SemaphoreType.DMA,
            pltpu.SemaphoreType.DMA,
            pltpu.SemaphoreType.DMA,
        ],
    )
    def body(idx_hbm, tok_hbm, pos_hbm, out_hbm, idx_all, tok_v, pos_v,
             sem_g, sem_p, sem_s):
        wid = lax.axis_index("s") * NC + lax.axis_index("c")
        t0 = wid * TB
        for b in range(B):
            pltpu.sync_copy(
                idx_hbm.at[pl.ds(b * T_sc + t0, TB)], idx_all.at[b])

        def drain_stores(n):
            for _ in range(n):
                pltpu.make_async_copy(
                    tok_hbm.at[pl.ds(0, Cq)], tok_v.at[0, 0], sem_s).wait()

        pltpu.sync_copy(pos_hbm.at[pl.ds(t0, Cq)], pos_v.at[0])
        for b in range(B):
            pltpu.async_copy(
                tok_hbm.at[idx_all.at[b, pl.ds(0, Cq)]],
                tok_v.at[0, b], sem_g)

        def chunk_body(ch, _):
            slot = lax.rem(ch, 2)
            nslot = 1 - slot

            @pl.when(ch >= 1)
            def _():
                drain_stores(B)

            @pl.when(ch + 1 < n_chunks)
            def _():
                for b in range(B):
                    pltpu.async_copy(
                        tok_hbm.at[idx_all.at[b, pl.ds((ch + 1) * Cq, Cq)]],
                        tok_v.at[nslot, b], sem_g)
                pltpu.async_copy(
                    pos_hbm.at[pl.ds(t0 + (ch + 1) * Cq, Cq)],
                    pos_v.at[nslot], sem_p)

            for _ in range(B):
                pltpu.make_async_copy(
                    tok_hbm.at[pl.ds(0, Cq)], tok_v.at[0, 0], sem_g).wait()

            @pl.when(ch >= 1)
            def _():
                pltpu.make_async_copy(
                    pos_hbm.at[pl.ds(0, Cq)], pos_v.at[0], sem_p).wait()

            def row_body(rr, _):
                for jb in range(D // _LANES):
                    off = jb * _LANES
                    pv = pos_v[slot, rr, pl.ds(off, _LANES)]
                    for b in range(B):
                        tok_v[slot, b, rr, pl.ds(off, _LANES)] = (
                            tok_v[slot, b, rr, pl.ds(off, _LANES)] + pv
                        )
                return 0

            lax.fori_loop(0, Cq, row_body, 0)
            for b in range(B):
                pltpu.async_copy(
                    tok_v.at[slot, b],
                    out_hbm.at[b, pl.ds(t0 + ch * Cq, Cq)],
                    sem_s)
            return 0

        lax.fori_loop(0, n_chunks, chunk_body, 0)
        drain_stores(B)

    return body


def kernel(idx, token_emb, pos_emb):
    B, T = idx.shape
    V, D = token_emb.shape
    info = plsc.get_sparse_core_info()
    NC, NS = info.num_cores, info.num_subcores
    T_sc = 3 * T // 4
    body = _make_kernel(B, T, V, D, NC, NS, Cq=8, T_sc=T_sc)
    idx_flat = idx[:, :T_sc].astype(jnp.int32).reshape(B * T_sc)
    out = body(idx_flat, token_emb, pos_emb)
    tc_part = (jnp.take(token_emb, idx[:, T_sc:], axis=0)
               + pos_emb[None, T_sc:, :])
    return lax.dynamic_update_slice(out, tc_part, (0, T_sc, 0))


# speedup vs baseline: 1.3584x; 1.3584x over previous
import functools

import jax
import jax.numpy as jnp
from jax import lax
from jax.experimental import pallas as pl
from jax.experimental.pallas import tpu as pltpu
from jax.experimental.pallas import tpu_sc as plsc

_LANES = 16


def _make_kernel(B, T, V, D, NC, NS, Cq, T_sc):
    NW = NC * NS
    TB = T_sc // NW
    n_chunks = TB // Cq
    mesh = plsc.VectorSubcoreMesh(core_axis_name="c", subcore_axis_name="s")

    @functools.partial(
        pl.kernel,
        mesh=mesh,
        out_type=jax.ShapeDtypeStruct((B, T, D), jnp.float32),
        scratch_types=[
            pltpu.VMEM((B, TB), jnp.int32),
            pltpu.VMEM((3, B, Cq, D), jnp.float32),
            pltpu.VMEM((3, Cq, D), jnp.float32),
            pltpu.SemaphoreType.DMA,
            pltpu.SemaphoreType.DMA,
            pltpu.SemaphoreType.DMA,
        ],
    )
    def body(idx_hbm, tok_hbm, pos_hbm, out_hbm, idx_all, tok_v, pos_v,
             sem_g, sem_p, sem_s):
        wid = lax.axis_index("s") * NC + lax.axis_index("c")
        t0 = wid * TB
        for b in range(B):
            pltpu.sync_copy(
                idx_hbm.at[pl.ds(b * T_sc + t0, TB)], idx_all.at[b])

        def drain_stores(n):
            for _ in range(n):
                pltpu.make_async_copy(
                    tok_hbm.at[pl.ds(0, Cq)], tok_v.at[0, 0], sem_s).wait()

        pltpu.sync_copy(pos_hbm.at[pl.ds(t0, Cq)], pos_v.at[0])
        for b in range(B):
            pltpu.async_copy(
                tok_hbm.at[idx_all.at[b, pl.ds(0, Cq)]],
                tok_v.at[0, b], sem_g)

        def chunk_body(ch, _):
            slot = lax.rem(ch, 3)
            nslot = lax.rem(ch + 1, 3)

            @pl.when(ch >= 2)
            def _():
                drain_stores(B)

            @pl.when(ch + 1 < n_chunks)
            def _():
                for b in range(B):
                    pltpu.async_copy(
                        tok_hbm.at[idx_all.at[b, pl.ds((ch + 1) * Cq, Cq)]],
                        tok_v.at[nslot, b], sem_g)
                pltpu.async_copy(
                    pos_hbm.at[pl.ds(t0 + (ch + 1) * Cq, Cq)],
                    pos_v.at[nslot], sem_p)

            for _ in range(B):
                pltpu.make_async_copy(
                    tok_hbm.at[pl.ds(0, Cq)], tok_v.at[0, 0], sem_g).wait()

            @pl.when(ch >= 1)
            def _():
                pltpu.make_async_copy(
                    pos_hbm.at[pl.ds(0, Cq)], pos_v.at[0], sem_p).wait()

            def row_body(rr, _):
                for jb in range(D // _LANES):
                    off = jb * _LANES
                    pv = pos_v[slot, rr, pl.ds(off, _LANES)]
                    for b in range(B):
                        tok_v[slot, b, rr, pl.ds(off, _LANES)] = (
                            tok_v[slot, b, rr, pl.ds(off, _LANES)] + pv
                        )
                return 0

            lax.fori_loop(0, Cq, row_body, 0)
            for b in range(B):
                pltpu.async_copy(
                    tok_v.at[slot, b],
                    out_hbm.at[b, pl.ds(t0 + ch * Cq, Cq)],
                    sem_s)
            return 0

        lax.fori_loop(0, n_chunks, chunk_body, 0)
        drain_stores(2 * B)

    return body


def kernel(idx, token_emb, pos_emb):
    B, T = idx.shape
    V, D = token_emb.shape
    info = plsc.get_sparse_core_info()
    NC, NS = info.num_cores, info.num_subcores
    T_sc = T
    body = _make_kernel(B, T, V, D, NC, NS, Cq=8, T_sc=T_sc)
    idx_flat = idx.astype(jnp.int32).reshape(B * T_sc)
    return body(idx_flat, token_emb, pos_emb)


# speedup vs baseline: 1.3671x; 1.0064x over previous
import functools

import jax
import jax.numpy as jnp
from jax import lax
from jax.experimental import pallas as pl
from jax.experimental.pallas import tpu as pltpu
from jax.experimental.pallas import tpu_sc as plsc

_LANES = 16


def _make_kernel(B, T, V, D, NC, NS, Cq):
    NW = NC * NS
    TB = T // NW
    n_chunks = TB // Cq
    R = B * Cq
    mesh = plsc.VectorSubcoreMesh(core_axis_name="c", subcore_axis_name="s")

    @functools.partial(
        pl.kernel,
        mesh=mesh,
        out_type=jax.ShapeDtypeStruct((B, T, D), jnp.float32),
        scratch_types=[
            pltpu.VMEM((n_chunks, R), jnp.int32),
            pltpu.VMEM((3, R, D), jnp.float32),
            pltpu.VMEM((3, Cq, D), jnp.float32),
            pltpu.SemaphoreType.DMA,
            pltpu.SemaphoreType.DMA,
            pltpu.SemaphoreType.DMA,
            pltpu.SemaphoreType.DMA,
        ],
    )
    def body(idx_hbm, tok_hbm, pos_hbm, out_hbm, idx_all, tok_v, pos_v,
             sem_i, sem_g, sem_p, sem_s):
        wid = lax.axis_index("s") * NC + lax.axis_index("c")
        t0 = wid * TB
        for b in range(B):
            for ch in range(n_chunks):
                pltpu.async_copy(
                    idx_hbm.at[pl.ds(b * T + t0 + ch * Cq, Cq)],
                    idx_all.at[ch, pl.ds(b * Cq, Cq)], sem_i)
        for b in range(B):
            for ch in range(n_chunks):
                pltpu.make_async_copy(
                    idx_hbm.at[pl.ds(0, Cq)],
                    idx_all.at[0, pl.ds(0, Cq)], sem_i).wait()

        def drain(sem, rows, n):
            for _ in range(n):
                pltpu.make_async_copy(
                    tok_hbm.at[pl.ds(0, rows)],
                    tok_v.at[0, pl.ds(0, rows)], sem).wait()

        pltpu.sync_copy(pos_hbm.at[pl.ds(t0, Cq)], pos_v.at[0])
        pltpu.async_copy(tok_hbm.at[idx_all.at[0]], tok_v.at[0], sem_g)

        def chunk_body(ch, _):
            slot = lax.rem(ch, 3)
            nslot = lax.rem(ch + 1, 3)

            @pl.when(ch >= 2)
            def _():
                drain(sem_s, Cq, B)

            @pl.when(ch + 1 < n_chunks)
            def _():
                pltpu.async_copy(
                    tok_hbm.at[idx_all.at[ch + 1]], tok_v.at[nslot], sem_g)
                pltpu.async_copy(
                    pos_hbm.at[pl.ds(t0 + (ch + 1) * Cq, Cq)],
                    pos_v.at[nslot], sem_p)

            drain(sem_g, R, 1)

            @pl.when(ch >= 1)
            def _():
                pltpu.make_async_copy(
                    pos_hbm.at[pl.ds(0, Cq)], pos_v.at[0], sem_p).wait()

            def row_body(rr, _):
                for jb in range(D // _LANES):
                    off = jb * _LANES
                    pv = pos_v[slot, rr, pl.ds(off, _LANES)]
                    for b in range(B):
                        tok_v[slot, b * Cq + rr, pl.ds(off, _LANES)] = (
                            tok_v[slot, b * Cq + rr, pl.ds(off, _LANES)] + pv
                        )
                return 0

            lax.fori_loop(0, Cq, row_body, 0)
            for b in range(B):
                pltpu.async_copy(
                    tok_v.at[slot, pl.ds(b * Cq, Cq)],
                    out_hbm.at[b, pl.ds(t0 + ch * Cq, Cq)],
                    sem_s)
            return 0

        lax.fori_loop(0, n_chunks, chunk_body, 0)
        drain(sem_s, Cq, 2 * B)

    return body


def kernel(idx, token_emb, pos_emb):
    B, T = idx.shape
    V, D = token_emb.shape
    info = plsc.get_sparse_core_info()
    NC, NS = info.num_cores, info.num_subcores
    body = _make_kernel(B, T, V, D, NC, NS, Cq=8)
    idx_flat = idx.astype(jnp.int32).reshape(B * T)
    return body(idx_flat, token_emb, pos_emb)
